# baseline (speedup 1.0000x reference)
import functools

import jax
import jax.numpy as jnp
from jax import lax
from jax.experimental import pallas as pl
from jax.experimental.pallas import tpu as pltpu
from jax.experimental.pallas import tpu_sc as plsc

B, C = 4, 128
N_FINE = 40962
N_COARSE = 10242
PATCH = 7

L = 16
NROWS = B * C
NF_PAD = 40968
NC_PAD = 10368
NB = 864
NIDX = PATCH * NC_PAD
N_CHUNKS = NC_PAD // L
ROWS_PER_TILE = NROWS // 32


def _sc_maxpool(xf, v2p_pad, npf):
    mesh = plsc.VectorSubcoreMesh(core_axis_name="c", subcore_axis_name="s")

    @functools.partial(
        pl.kernel,
        mesh=mesh,
        compiler_params=pltpu.CompilerParams(needs_layout_passes=False),
        out_type=jax.ShapeDtypeStruct((NROWS * NC_PAD,), jnp.float32),
        scratch_types=[
            pltpu.VMEM((NIDX,), jnp.int32),
        ],
    )
    def k(xf_hbm, v2p_hbm, npf_hbm, out_hbm, idxT):
        wid = lax.axis_index("s") * 2 + lax.axis_index("c")

        def phase1(v2pbuf, npbuf):
            pltpu.sync_copy(v2p_hbm, v2pbuf)
            i7 = lax.iota(jnp.int32, L) * PATCH

            def chunk_body(cc, carry):
                pltpu.sync_copy(npf_hbm.at[pl.ds(cc * (NB * PATCH), NB * PATCH)],
                                npbuf)
                for p in range(PATCH):

                    def comp_body(ci, carry2, p=p):
                        iv = plsc.load_gather(npbuf, [i7 + (ci * (L * PATCH) + p)])
                        fv = plsc.load_gather(v2pbuf, [iv])
                        idxT[pl.ds(p * NC_PAD + cc * NB + ci * L, L)] = fv
                        return carry2

                    lax.fori_loop(0, NB // L, comp_body, 0)
                return carry

            lax.fori_loop(0, NC_PAD // NB, chunk_body, 0)

        pl.run_scoped(phase1,
                      pltpu.VMEM((NF_PAD,), jnp.int32),
                      pltpu.VMEM((NB * PATCH,), jnp.int32))

        def phase2(xrow, outbuf):
            def row_body(kk, carry):
                r = wid * ROWS_PER_TILE + kk
                start = r * N_FINE
                shift = lax.rem(start, 8)
                a0 = pl.multiple_of(start - shift, 8)
                pltpu.sync_copy(xf_hbm.at[pl.ds(a0, NF_PAD)], xrow)

                def chunk_body(ci, carry2):
                    i0 = ci * L
                    acc = plsc.load_gather(
                        xrow, [idxT[pl.ds(i0, L)] + shift])
                    for p in range(1, PATCH):
                        acc = jnp.maximum(acc, plsc.load_gather(
                            xrow, [idxT[pl.ds(p * NC_PAD + i0, L)] + shift]))
                    outbuf[pl.ds(i0, L)] = acc
                    return carry2

                lax.fori_loop(0, N_CHUNKS, chunk_body, 0)
                pltpu.sync_copy(outbuf, out_hbm.at[pl.ds(r * NC_PAD, NC_PAD)])
                return carry

            lax.fori_loop(0, ROWS_PER_TILE, row_body, 0)

        pl.run_scoped(phase2,
                      pltpu.VMEM((NF_PAD,), jnp.float32),
                      pltpu.VMEM((NC_PAD,), jnp.float32))

    return k(xf, v2p_pad, npf)


def kernel(x, vertices_to_prev_lvl, neighbor_patches):
    xf = x.reshape(-1)
    v2p_pad = jnp.pad(vertices_to_prev_lvl, (0, NF_PAD - N_FINE))
    npf = jnp.pad(neighbor_patches,
                  ((0, NC_PAD - N_COARSE), (0, 0))).reshape(-1)

    out_pad = _sc_maxpool(xf, v2p_pad, npf).reshape(NROWS, NC_PAD)
    return out_pad[:, :N_COARSE].reshape(B, C, N_COARSE)
